# Initial kernel scaffold; baseline (speedup 1.0000x reference)
import jax
import jax.numpy as jnp
from jax.experimental import pallas as pl


def kernel(X, X_char, word_table, char_table, W3, W5):
    raise NotImplementedError("write your pallas kernel here")


# speedup vs baseline: 4.8956x; 4.8956x over previous
import functools

import jax
import jax.numpy as jnp
from jax import lax
from jax.experimental import pallas as pl
from jax.experimental.pallas import tpu as pltpu
from jax.experimental.pallas import tpu_sc as plsc

_NC = 2
_NS = 16
_NW = _NC * _NS

_D_CHAR = 16
_L_CHARS = 16
_D_WORD = 128


def _sc_gather(num_rows, row_dim, chunk):
    per_w = num_rows // _NW
    n_chunks = per_w // chunk
    mesh = plsc.VectorSubcoreMesh(core_axis_name="c", subcore_axis_name="s")

    @functools.partial(
        pl.kernel,
        out_type=jax.ShapeDtypeStruct((num_rows, row_dim), jnp.float32),
        mesh=mesh,
        compiler_params=pltpu.CompilerParams(use_tc_tiling_on_sc=False),
        scratch_types=[
            pltpu.VMEM((chunk,), jnp.int32),
            pltpu.VMEM((chunk, row_dim), jnp.float32),
            pltpu.SemaphoreType.DMA,
        ],
    )
    def gather(idx_hbm, table_hbm, out_hbm, idx_c, rows_v, sem):
        wid = lax.axis_index("s") * _NC + lax.axis_index("c")
        base = wid * per_w
        for c in range(n_chunks):
            off = base + c * chunk
            pltpu.sync_copy(idx_hbm.at[pl.ds(off, chunk)], idx_c)
            pltpu.async_copy(table_hbm.at[idx_c], rows_v, sem).wait()
            pltpu.sync_copy(rows_v, out_hbm.at[pl.ds(off, chunk)])

    return gather


def _band(W, k):
    O = W.shape[0]
    T = jnp.transpose(W, (2, 1, 0))
    Tz = jnp.concatenate([T, jnp.zeros((1, _D_CHAR, O), W.dtype)], axis=0)
    t = jnp.arange(_L_CHARS)[:, None]
    p = jnp.arange(_L_CHARS)[None, :]
    dk = p - t + k // 2
    idx = jnp.where((dk >= 0) & (dk < k), dk, k)
    return Tz[idx]


_TN = 256


def _conv_body(ce_ref, wv_ref, wb_ref, out_ref):
    acc = jnp.dot(ce_ref[...], wb_ref[...],
                  preferred_element_type=jnp.float32)
    m = acc[:, :256]
    for t in range(1, _L_CHARS):
        m = jnp.maximum(m, acc[:, 256 * t:256 * (t + 1)])
    ch = jnp.maximum(m[:, :_D_WORD], m[:, _D_WORD:])
    out_ref[...] = wv_ref[...] + jnp.maximum(ch, jnp.float32(0))


def kernel(X, X_char, word_table, char_table, W3, W5):
    B, S = X.shape
    N = B * S
    flat_words = X.reshape(N).astype(jnp.int32)
    flat_chars = X_char.reshape(N * _L_CHARS).astype(jnp.int32)

    word_vecs = _sc_gather(N, _D_WORD, 400)(flat_words, word_table)
    char_emb = _sc_gather(N * _L_CHARS, _D_CHAR, 3200)(flat_chars, char_table)
    ce = char_emb.reshape(N, _L_CHARS * _D_CHAR)

    Wb = jnp.concatenate([_band(W3, 3), _band(W5, 5)], axis=-1)
    Wb = jnp.transpose(Wb, (1, 2, 0, 3)).reshape(256, _L_CHARS * 256)

    out = pl.pallas_call(
        _conv_body,
        grid=(N // _TN,),
        in_specs=[
            pl.BlockSpec((_TN, 256), lambda i: (i, 0)),
            pl.BlockSpec((_TN, _D_WORD), lambda i: (i, 0)),
            pl.BlockSpec((256, _L_CHARS * 256), lambda i: (0, 0)),
        ],
        out_specs=pl.BlockSpec((_TN, _D_WORD), lambda i: (i, 0)),
        out_shape=jax.ShapeDtypeStruct((N, _D_WORD), jnp.float32),
    )(ce, word_vecs, Wb)

    return out.reshape(B, S, _D_WORD)


# speedup vs baseline: 4.9174x; 1.0045x over previous
import functools

import jax
import jax.numpy as jnp
from jax import lax
from jax.experimental import pallas as pl
from jax.experimental.pallas import tpu as pltpu
from jax.experimental.pallas import tpu_sc as plsc

_NC = 2
_NS = 16
_NW = _NC * _NS

_D_CHAR = 16
_L_CHARS = 16
_D_WORD = 128


def _sc_gather(num_rows, row_dim, chunk):
    per_w = num_rows // _NW
    n_chunks = per_w // chunk
    mesh = plsc.VectorSubcoreMesh(core_axis_name="c", subcore_axis_name="s")

    @functools.partial(
        pl.kernel,
        out_type=jax.ShapeDtypeStruct((num_rows, row_dim), jnp.float32),
        mesh=mesh,
        compiler_params=pltpu.CompilerParams(use_tc_tiling_on_sc=False),
        scratch_types=[
            pltpu.VMEM((chunk,), jnp.int32),
            pltpu.VMEM((chunk, row_dim), jnp.float32),
            pltpu.SemaphoreType.DMA,
        ],
    )
    def gather(idx_hbm, table_hbm, out_hbm, idx_c, rows_v, sem):
        wid = lax.axis_index("s") * _NC + lax.axis_index("c")
        base = wid * per_w
        for c in range(n_chunks):
            off = base + c * chunk
            pltpu.sync_copy(idx_hbm.at[pl.ds(off, chunk)], idx_c)
            pltpu.async_copy(table_hbm.at[idx_c], rows_v, sem).wait()
            pltpu.sync_copy(rows_v, out_hbm.at[pl.ds(off, chunk)])

    return gather


def _band(W, k):
    O = W.shape[0]
    T = jnp.transpose(W, (2, 1, 0))
    Tz = jnp.concatenate([T, jnp.zeros((1, _D_CHAR, O), W.dtype)], axis=0)
    t = jnp.arange(_L_CHARS)[:, None]
    p = jnp.arange(_L_CHARS)[None, :]
    dk = p - t + k // 2
    idx = jnp.where((dk >= 0) & (dk < k), dk, k)
    return Tz[idx]


_TN = 256


def _conv_body(ce_ref, wv_ref, wb_ref, out_ref):
    acc = jnp.dot(ce_ref[...].astype(jnp.bfloat16), wb_ref[...],
                  preferred_element_type=jnp.float32)
    m = acc[:, :256]
    for t in range(1, _L_CHARS):
        m = jnp.maximum(m, acc[:, 256 * t:256 * (t + 1)])
    ch = jnp.maximum(m[:, :_D_WORD], m[:, _D_WORD:])
    out_ref[...] = wv_ref[...] + jnp.maximum(ch, jnp.float32(0))


def kernel(X, X_char, word_table, char_table, W3, W5):
    B, S = X.shape
    N = B * S
    flat_words = X.reshape(N).astype(jnp.int32)
    flat_chars = X_char.reshape(N * _L_CHARS).astype(jnp.int32)

    word_vecs = _sc_gather(N, _D_WORD, 400)(flat_words, word_table)
    char_emb = _sc_gather(N * _L_CHARS, _D_CHAR, 3200)(flat_chars, char_table)
    ce = char_emb.reshape(N, _L_CHARS * _D_CHAR)

    Wb = jnp.concatenate([_band(W3, 3), _band(W5, 5)], axis=-1)
    Wb = jnp.transpose(Wb, (1, 2, 0, 3)).reshape(256, _L_CHARS * 256)
    Wb = Wb.astype(jnp.bfloat16)

    out = pl.pallas_call(
        _conv_body,
        grid=(N // _TN,),
        in_specs=[
            pl.BlockSpec((_TN, 256), lambda i: (i, 0)),
            pl.BlockSpec((_TN, _D_WORD), lambda i: (i, 0)),
            pl.BlockSpec((256, _L_CHARS * 256), lambda i: (0, 0)),
        ],
        out_specs=pl.BlockSpec((_TN, _D_WORD), lambda i: (i, 0)),
        out_shape=jax.ShapeDtypeStruct((N, _D_WORD), jnp.float32),
    )(ce, word_vecs, Wb)

    return out.reshape(B, S, _D_WORD)


# speedup vs baseline: 4.9272x; 1.0020x over previous
import functools

import jax
import jax.numpy as jnp
from jax import lax
from jax.experimental import pallas as pl
from jax.experimental.pallas import tpu as pltpu
from jax.experimental.pallas import tpu_sc as plsc

_NC = 2
_NS = 16
_NW = _NC * _NS

_D_CHAR = 16
_L_CHARS = 16
_D_WORD = 128


def _sc_gather(num_rows, row_dim, chunk, tc_tiling):
    per_w = num_rows // _NW
    n_chunks = per_w // chunk
    mesh = plsc.VectorSubcoreMesh(core_axis_name="c", subcore_axis_name="s")

    @functools.partial(
        pl.kernel,
        out_type=jax.ShapeDtypeStruct((num_rows, row_dim), jnp.float32),
        mesh=mesh,
        compiler_params=pltpu.CompilerParams(use_tc_tiling_on_sc=tc_tiling),
        scratch_types=[
            pltpu.VMEM((chunk,), jnp.int32),
            pltpu.VMEM((chunk, row_dim), jnp.float32),
            pltpu.SemaphoreType.DMA,
        ],
    )
    def gather(idx_hbm, table_hbm, out_hbm, idx_c, rows_v, sem):
        wid = lax.axis_index("s") * _NC + lax.axis_index("c")
        base = wid * per_w
        for c in range(n_chunks):
            off = base + c * chunk
            pltpu.sync_copy(idx_hbm.at[pl.ds(off, chunk)], idx_c)
            pltpu.async_copy(table_hbm.at[idx_c], rows_v, sem).wait()
            pltpu.sync_copy(rows_v, out_hbm.at[pl.ds(off, chunk)])

    return gather


def _band(W, k):
    O = W.shape[0]
    T = jnp.transpose(W, (2, 1, 0))
    Tz = jnp.concatenate([T, jnp.zeros((1, _D_CHAR, O), W.dtype)], axis=0)
    t = jnp.arange(_L_CHARS)[:, None]
    p = jnp.arange(_L_CHARS)[None, :]
    dk = p - t + k // 2
    idx = jnp.where((dk >= 0) & (dk < k), dk, k)
    return Tz[idx]


_TN = 256


def _conv_body(ce_ref, wv_ref, wb_ref, out_ref):
    acc = jnp.dot(ce_ref[...].astype(jnp.bfloat16), wb_ref[...],
                  preferred_element_type=jnp.float32)
    m = acc[:, :256]
    for t in range(1, _L_CHARS):
        m = jnp.maximum(m, acc[:, 256 * t:256 * (t + 1)])
    ch = jnp.maximum(m[:, :_D_WORD], m[:, _D_WORD:])
    out_ref[...] = wv_ref[...] + jnp.maximum(ch, jnp.float32(0))


def kernel(X, X_char, word_table, char_table, W3, W5):
    B, S = X.shape
    N = B * S
    flat_words = X.reshape(N).astype(jnp.int32)
    flat_chars = X_char.reshape(N * _L_CHARS).astype(jnp.int32)

    word_vecs = _sc_gather(N, _D_WORD, 400, True)(flat_words, word_table)
    char_emb = _sc_gather(N * _L_CHARS, _D_CHAR, 3200, False)(flat_chars, char_table)
    ce = char_emb.reshape(N, _L_CHARS * _D_CHAR)

    Wb = jnp.concatenate([_band(W3, 3), _band(W5, 5)], axis=-1)
    Wb = jnp.transpose(Wb, (1, 2, 0, 3)).reshape(256, _L_CHARS * 256)
    Wb = Wb.astype(jnp.bfloat16)

    out = pl.pallas_call(
        _conv_body,
        grid=(N // _TN,),
        in_specs=[
            pl.BlockSpec((_TN, 256), lambda i: (i, 0)),
            pl.BlockSpec((_TN, _D_WORD), lambda i: (i, 0)),
            pl.BlockSpec((256, _L_CHARS * 256), lambda i: (0, 0)),
        ],
        out_specs=pl.BlockSpec((_TN, _D_WORD), lambda i: (i, 0)),
        out_shape=jax.ShapeDtypeStruct((N, _D_WORD), jnp.float32),
    )(ce, word_vecs, Wb)

    return out.reshape(B, S, _D_WORD)


# speedup vs baseline: 5.6379x; 1.1442x over previous
import functools

import jax
import jax.numpy as jnp
from jax import lax
from jax.experimental import pallas as pl
from jax.experimental.pallas import tpu as pltpu
from jax.experimental.pallas import tpu_sc as plsc

_NC = 2
_NS = 16
_NW = _NC * _NS

_D_CHAR = 16
_L_CHARS = 16
_D_WORD = 128


def _sc_gather(num_rows, row_dim, chunk, tc_tiling):
    per_w = num_rows // _NW
    n_chunks = per_w // chunk
    mesh = plsc.VectorSubcoreMesh(core_axis_name="c", subcore_axis_name="s")

    @functools.partial(
        pl.kernel,
        out_type=jax.ShapeDtypeStruct((num_rows, row_dim), jnp.float32),
        mesh=mesh,
        compiler_params=pltpu.CompilerParams(use_tc_tiling_on_sc=tc_tiling),
        scratch_types=[
            pltpu.VMEM((chunk,), jnp.int32),
            pltpu.VMEM((chunk,), jnp.int32),
            pltpu.VMEM((chunk, row_dim), jnp.float32),
            pltpu.VMEM((chunk, row_dim), jnp.float32),
            pltpu.SemaphoreType.DMA,
            pltpu.SemaphoreType.DMA,
            pltpu.SemaphoreType.DMA,
            pltpu.SemaphoreType.DMA,
        ],
    )
    def gather(idx_hbm, table_hbm, out_hbm, idx0, idx1, rows0, rows1,
               gsem0, gsem1, osem0, osem1):
        wid = lax.axis_index("s") * _NC + lax.axis_index("c")
        base = wid * per_w
        idx_b = [idx0, idx1]
        rows_b = [rows0, rows1]
        gsem = [gsem0, gsem1]
        osem = [osem0, osem1]
        h_g = [None, None]
        h_o = [None, None]
        pltpu.sync_copy(idx_hbm.at[pl.ds(base, chunk)], idx_b[0])
        h_g[0] = pltpu.async_copy(
            table_hbm.at[idx_b[0]], rows_b[0], gsem[0])
        for c in range(n_chunks):
            cur, nxt = c % 2, (c + 1) % 2
            h_g[cur].wait()
            if c + 1 < n_chunks:
                pltpu.sync_copy(
                    idx_hbm.at[pl.ds(base + (c + 1) * chunk, chunk)],
                    idx_b[nxt])
                if c >= 1:
                    h_o[nxt].wait()
                h_g[nxt] = pltpu.async_copy(
                    table_hbm.at[idx_b[nxt]], rows_b[nxt], gsem[nxt])
            h_o[cur] = pltpu.async_copy(
                rows_b[cur], out_hbm.at[pl.ds(base + c * chunk, chunk)],
                osem[cur])
        h_o[(n_chunks - 1) % 2].wait()
        if n_chunks > 1:
            h_o[n_chunks % 2].wait()

    return gather


def _band(W, k):
    O = W.shape[0]
    T = jnp.transpose(W, (2, 1, 0))
    Tz = jnp.concatenate([T, jnp.zeros((1, _D_CHAR, O), W.dtype)], axis=0)
    t = jnp.arange(_L_CHARS)[:, None]
    p = jnp.arange(_L_CHARS)[None, :]
    dk = p - t + k // 2
    idx = jnp.where((dk >= 0) & (dk < k), dk, k)
    return Tz[idx]


_TN = 256


def _conv_body(ce_ref, wv_ref, wb_ref, out_ref):
    acc = jnp.dot(ce_ref[...].astype(jnp.bfloat16), wb_ref[...],
                  preferred_element_type=jnp.float32)
    m = acc[:, :256]
    for t in range(1, _L_CHARS):
        m = jnp.maximum(m, acc[:, 256 * t:256 * (t + 1)])
    ch = jnp.maximum(m[:, :_D_WORD], m[:, _D_WORD:])
    out_ref[...] = wv_ref[...] + jnp.maximum(ch, jnp.float32(0))


def kernel(X, X_char, word_table, char_table, W3, W5):
    B, S = X.shape
    N = B * S
    flat_words = X.reshape(N).astype(jnp.int32)
    flat_chars = X_char.reshape(N * _L_CHARS).astype(jnp.int32)
    pair_ids = flat_chars[0::2] * 128 + flat_chars[1::2]
    pair_table = jnp.concatenate(
        [jnp.repeat(char_table, 128, axis=0),
         jnp.tile(char_table, (128, 1))], axis=1)

    word_vecs = _sc_gather(N, _D_WORD, 400, True)(flat_words, word_table)
    char_emb = _sc_gather(N * 8, 2 * _D_CHAR, 1600, False)(
        pair_ids, pair_table)
    ce = char_emb.reshape(N, _L_CHARS * _D_CHAR)

    Wb = jnp.concatenate([_band(W3, 3), _band(W5, 5)], axis=-1)
    Wb = jnp.transpose(Wb, (1, 2, 0, 3)).reshape(256, _L_CHARS * 256)
    Wb = Wb.astype(jnp.bfloat16)

    out = pl.pallas_call(
        _conv_body,
        grid=(N // _TN,),
        in_specs=[
            pl.BlockSpec((_TN, 256), lambda i: (i, 0)),
            pl.BlockSpec((_TN, _D_WORD), lambda i: (i, 0)),
            pl.BlockSpec((256, _L_CHARS * 256), lambda i: (0, 0)),
        ],
        out_specs=pl.BlockSpec((_TN, _D_WORD), lambda i: (i, 0)),
        out_shape=jax.ShapeDtypeStruct((N, _D_WORD), jnp.float32),
    )(ce, word_vecs, Wb)

    return out.reshape(B, S, _D_WORD)


# speedup vs baseline: 5.9190x; 1.0499x over previous
import functools

import jax
import jax.numpy as jnp
from jax import lax
from jax.experimental import pallas as pl
from jax.experimental.pallas import tpu as pltpu
from jax.experimental.pallas import tpu_sc as plsc

_NC = 2
_NS = 16
_NW = _NC * _NS

_D_CHAR = 16
_L_CHARS = 16
_D_WORD = 128
_TN = 400


def _sc_word_gather(num_rows, chunk):
    per_w = num_rows // _NW
    n_chunks = per_w // chunk
    mesh = plsc.VectorSubcoreMesh(core_axis_name="c", subcore_axis_name="s")

    @functools.partial(
        pl.kernel,
        out_type=jax.ShapeDtypeStruct((num_rows, _D_WORD), jnp.float32),
        mesh=mesh,
        compiler_params=pltpu.CompilerParams(use_tc_tiling_on_sc=True),
        scratch_types=[
            pltpu.VMEM((chunk,), jnp.int32),
            pltpu.VMEM((chunk,), jnp.int32),
            pltpu.VMEM((chunk, _D_WORD), jnp.float32),
            pltpu.VMEM((chunk, _D_WORD), jnp.float32),
            pltpu.SemaphoreType.DMA,
            pltpu.SemaphoreType.DMA,
            pltpu.SemaphoreType.DMA,
            pltpu.SemaphoreType.DMA,
        ],
    )
    def gather(idx_hbm, table_hbm, out_hbm, idx0, idx1, rows0, rows1,
               gsem0, gsem1, osem0, osem1):
        wid = lax.axis_index("s") * _NC + lax.axis_index("c")
        base = wid * per_w
        idx_b, rows_b = [idx0, idx1], [rows0, rows1]
        gsem, osem = [gsem0, gsem1], [osem0, osem1]
        h_g = [None, None]
        h_o = [None, None]
        pltpu.sync_copy(idx_hbm.at[pl.ds(base, chunk)], idx_b[0])
        h_g[0] = pltpu.async_copy(table_hbm.at[idx_b[0]], rows_b[0], gsem[0])
        for c in range(n_chunks):
            cur, nxt = c % 2, (c + 1) % 2
            h_g[cur].wait()
            if c + 1 < n_chunks:
                pltpu.sync_copy(
                    idx_hbm.at[pl.ds(base + (c + 1) * chunk, chunk)],
                    idx_b[nxt])
                if c >= 1:
                    h_o[nxt].wait()
                h_g[nxt] = pltpu.async_copy(
                    table_hbm.at[idx_b[nxt]], rows_b[nxt], gsem[nxt])
            h_o[cur] = pltpu.async_copy(
                rows_b[cur], out_hbm.at[pl.ds(base + c * chunk, chunk)],
                osem[cur])
        h_o[(n_chunks - 1) % 2].wait()
        if n_chunks > 1:
            h_o[n_chunks % 2].wait()

    return gather


def _sc_char_gather(num_pairs, chunk):
    per_w = num_pairs // _NW
    n_chunks = per_w // chunk
    mesh = plsc.VectorSubcoreMesh(core_axis_name="c", subcore_axis_name="s")

    @functools.partial(
        pl.kernel,
        out_type=jax.ShapeDtypeStruct((num_pairs, 2 * _D_CHAR), jnp.float32),
        mesh=mesh,
        compiler_params=pltpu.CompilerParams(use_tc_tiling_on_sc=False),
        scratch_types=[
            pltpu.VMEM((chunk,), jnp.int32),
            pltpu.VMEM((chunk,), jnp.int32),
            pltpu.VMEM((chunk,), jnp.int32),
            pltpu.VMEM((chunk,), jnp.int32),
            pltpu.VMEM((chunk,), jnp.int32),
            pltpu.VMEM((chunk,), jnp.int32),
            pltpu.VMEM((chunk, 2 * _D_CHAR), jnp.float32),
            pltpu.VMEM((chunk, 2 * _D_CHAR), jnp.float32),
            pltpu.SemaphoreType.DMA,
            pltpu.SemaphoreType.DMA,
            pltpu.SemaphoreType.DMA,
            pltpu.SemaphoreType.DMA,
        ],
    )
    def gather(ev_hbm, od_hbm, table_hbm, out_hbm, ev0, ev1, od0, od1,
               idx0, idx1, rows0, rows1, gsem0, gsem1, osem0, osem1):
        wid = lax.axis_index("s") * _NC + lax.axis_index("c")
        base = wid * per_w
        ev_b, od_b = [ev0, ev1], [od0, od1]
        idx_b, rows_b = [idx0, idx1], [rows0, rows1]
        gsem, osem = [gsem0, gsem1], [osem0, osem1]
        h_g = [None, None]
        h_o = [None, None]

        def load_and_pair(c, slot):
            off = base + c * chunk
            pltpu.sync_copy(ev_hbm.at[pl.ds(off, chunk)], ev_b[slot])
            pltpu.sync_copy(od_hbm.at[pl.ds(off, chunk)], od_b[slot])

            def body(i, _):
                e = ev_b[slot][pl.ds(i * 16, 16)]
                o = od_b[slot][pl.ds(i * 16, 16)]
                idx_b[slot][pl.ds(i * 16, 16)] = e * 128 + o
                return 0

            lax.fori_loop(0, chunk // 16, body, 0)
            return pltpu.async_copy(
                table_hbm.at[idx_b[slot]], rows_b[slot], gsem[slot])

        h_g[0] = load_and_pair(0, 0)
        for c in range(n_chunks):
            cur, nxt = c % 2, (c + 1) % 2
            h_g[cur].wait()
            if c + 1 < n_chunks:
                if c >= 1:
                    h_o[nxt].wait()
                h_g[nxt] = load_and_pair(c + 1, nxt)
            h_o[cur] = pltpu.async_copy(
                rows_b[cur], out_hbm.at[pl.ds(base + c * chunk, chunk)],
                osem[cur])
        h_o[(n_chunks - 1) % 2].wait()
        if n_chunks > 1:
            h_o[n_chunks % 2].wait()

    return gather


def _band(W, k):
    O = W.shape[0]
    T = jnp.transpose(W, (2, 1, 0))
    Tz = jnp.concatenate([T, jnp.zeros((1, _D_CHAR, O), W.dtype)], axis=0)
    t = jnp.arange(_L_CHARS)[:, None]
    p = jnp.arange(_L_CHARS)[None, :]
    dk = p - t + k // 2
    idx = jnp.where((dk >= 0) & (dk < k), dk, k)
    return Tz[idx]


def _conv_body(ce_ref, wv_ref, w1_ref, w2_ref, out_ref):
    a1 = ce_ref[: _TN, :].astype(jnp.bfloat16)
    a2 = ce_ref[_TN:, :].astype(jnp.bfloat16)
    acc = jnp.dot(a1, w1_ref[...], preferred_element_type=jnp.float32)
    acc = acc + jnp.dot(a2, w2_ref[...], preferred_element_type=jnp.float32)
    m = acc[:, :256]
    for t in range(1, _L_CHARS):
        m = jnp.maximum(m, acc[:, 256 * t:256 * (t + 1)])
    ch = jnp.maximum(m[:, :_D_WORD], m[:, _D_WORD:])
    res = wv_ref[...] + jnp.maximum(ch, jnp.float32(0))
    for b in range(_TN // 50):
        out_ref[b] = res[b * 50:(b + 1) * 50, :]


def kernel(X, X_char, word_table, char_table, W3, W5):
    B, S = X.shape
    N = B * S
    n_blk = N // _TN
    flat_words = X.reshape(N).astype(jnp.int32)
    chars_perm = X_char.astype(jnp.int32).reshape(n_blk, _TN, 2, 4, 2)
    chars_perm = jnp.transpose(chars_perm, (4, 0, 2, 1, 3))
    chars_ev = chars_perm[0].reshape(-1)
    chars_od = chars_perm[1].reshape(-1)

    pair_table = jnp.concatenate(
        [jnp.repeat(char_table, 128, axis=0),
         jnp.tile(char_table, (128, 1))], axis=1)

    word_vecs = _sc_word_gather(N, 400)(flat_words, word_table)
    char_emb = _sc_char_gather(N * 8, 1600)(chars_ev, chars_od, pair_table)
    ce = char_emb.reshape(N * 2, _D_WORD)

    Wb = jnp.concatenate([_band(W3, 3), _band(W5, 5)], axis=-1)
    Wb = jnp.transpose(Wb, (1, 2, 0, 3)).reshape(256, _L_CHARS * 256)
    Wb = Wb.astype(jnp.bfloat16)
    W1, W2 = Wb[:_D_WORD], Wb[_D_WORD:]

    out = pl.pallas_call(
        _conv_body,
        grid=(n_blk,),
        in_specs=[
            pl.BlockSpec((2 * _TN, _D_WORD), lambda i: (i, 0)),
            pl.BlockSpec((_TN, _D_WORD), lambda i: (i, 0)),
            pl.BlockSpec((_D_WORD, _L_CHARS * 256), lambda i: (0, 0)),
            pl.BlockSpec((_D_WORD, _L_CHARS * 256), lambda i: (0, 0)),
        ],
        out_specs=pl.BlockSpec((_TN // 50, S, _D_WORD), lambda i: (i, 0, 0)),
        out_shape=jax.ShapeDtypeStruct((B, S, _D_WORD), jnp.float32),
    )(ce, word_vecs, W1, W2)

    return out


# speedup vs baseline: 6.5586x; 1.1081x over previous
import functools

import jax
import jax.numpy as jnp
from jax import lax
from jax.experimental import pallas as pl
from jax.experimental.pallas import tpu as pltpu
from jax.experimental.pallas import tpu_sc as plsc

_NC = 2
_NS = 16
_NW = _NC * _NS

_D_CHAR = 16
_L_CHARS = 16
_D_WORD = 128
_TN = 400


def _sc_gather(num_rows, row_dim, chunk, tc_tiling):
    per_w = num_rows // _NW
    n_chunks = per_w // chunk
    mesh = plsc.VectorSubcoreMesh(core_axis_name="c", subcore_axis_name="s")

    @functools.partial(
        pl.kernel,
        out_type=jax.ShapeDtypeStruct((num_rows, row_dim), jnp.float32),
        mesh=mesh,
        compiler_params=pltpu.CompilerParams(use_tc_tiling_on_sc=tc_tiling),
        scratch_types=[
            pltpu.VMEM((chunk,), jnp.int32),
            pltpu.VMEM((chunk,), jnp.int32),
            pltpu.VMEM((chunk, row_dim), jnp.float32),
            pltpu.VMEM((chunk, row_dim), jnp.float32),
            pltpu.SemaphoreType.DMA,
            pltpu.SemaphoreType.DMA,
            pltpu.SemaphoreType.DMA,
            pltpu.SemaphoreType.DMA,
        ],
    )
    def gather(idx_hbm, table_hbm, out_hbm, idx0, idx1, rows0, rows1,
               gsem0, gsem1, osem0, osem1):
        wid = lax.axis_index("s") * _NC + lax.axis_index("c")
        base = wid * per_w
        idx_b, rows_b = [idx0, idx1], [rows0, rows1]
        gsem, osem = [gsem0, gsem1], [osem0, osem1]
        h_g = [None, None]
        h_o = [None, None]
        pltpu.sync_copy(idx_hbm.at[pl.ds(base, chunk)], idx_b[0])
        h_g[0] = pltpu.async_copy(table_hbm.at[idx_b[0]], rows_b[0], gsem[0])
        for c in range(n_chunks):
            cur, nxt = c % 2, (c + 1) % 2
            h_g[cur].wait()
            if c + 1 < n_chunks:
                pltpu.sync_copy(
                    idx_hbm.at[pl.ds(base + (c + 1) * chunk, chunk)],
                    idx_b[nxt])
                if c >= 1:
                    h_o[nxt].wait()
                h_g[nxt] = pltpu.async_copy(
                    table_hbm.at[idx_b[nxt]], rows_b[nxt], gsem[nxt])
            h_o[cur] = pltpu.async_copy(
                rows_b[cur], out_hbm.at[pl.ds(base + c * chunk, chunk)],
                osem[cur])
        h_o[(n_chunks - 1) % 2].wait()
        if n_chunks > 1:
            h_o[n_chunks % 2].wait()

    return gather


def _band(W, k):
    O = W.shape[0]
    T = jnp.transpose(W, (2, 1, 0))
    Tz = jnp.concatenate([T, jnp.zeros((1, _D_CHAR, O), W.dtype)], axis=0)
    t = jnp.arange(_L_CHARS)[:, None]
    p = jnp.arange(_L_CHARS)[None, :]
    dk = p - t + k // 2
    idx = jnp.where((dk >= 0) & (dk < k), dk, k)
    return Tz[idx]


def _conv_body(ce_ref, wv_ref, wb_ref, out_ref):
    x2 = ce_ref[...].astype(jnp.bfloat16)
    x = x2.reshape(_TN, 2 * _D_WORD)
    acc = jnp.dot(x, wb_ref[...], preferred_element_type=jnp.float32)
    m = acc[:, :256]
    for t in range(1, _L_CHARS):
        m = jnp.maximum(m, acc[:, 256 * t:256 * (t + 1)])
    ch = jnp.maximum(m[:, :_D_WORD], m[:, _D_WORD:])
    res = wv_ref[...] + jnp.maximum(ch, jnp.float32(0))
    for b in range(_TN // 50):
        out_ref[b] = res[b * 50:(b + 1) * 50, :]


def kernel(X, X_char, word_table, char_table, W3, W5):
    B, S = X.shape
    N = B * S
    n_blk = N // _TN
    flat_words = X.reshape(N).astype(jnp.int32)

    chars8 = X_char.astype(jnp.int8).reshape(N * 8, 2)
    pids = lax.bitcast_convert_type(chars8, jnp.int16).astype(jnp.int32)

    padded = jnp.pad(char_table, ((0, 128), (0, 0)))
    pair_table = jnp.concatenate(
        [jnp.tile(padded, (128, 1)),
         jnp.repeat(char_table, 256, axis=0)], axis=1)

    word_vecs = _sc_gather(N, _D_WORD, 400, True)(flat_words, word_table)
    char_emb = _sc_gather(N * 8, 2 * _D_CHAR, 1600, False)(pids, pair_table)
    ce = char_emb.reshape(N * 2, _D_WORD)

    Wb = jnp.concatenate([_band(W3, 3), _band(W5, 5)], axis=-1)
    Wb = jnp.transpose(Wb, (1, 2, 0, 3)).reshape(256, _L_CHARS * 256)
    Wb = Wb.astype(jnp.bfloat16)

    out = pl.pallas_call(
        _conv_body,
        grid=(n_blk,),
        in_specs=[
            pl.BlockSpec((2 * _TN, _D_WORD), lambda i: (i, 0)),
            pl.BlockSpec((_TN, _D_WORD), lambda i: (i, 0)),
            pl.BlockSpec((2 * _D_WORD, _L_CHARS * 256), lambda i: (0, 0)),
        ],
        out_specs=pl.BlockSpec((_TN // 50, S, _D_WORD), lambda i: (i, 0, 0)),
        out_shape=jax.ShapeDtypeStruct((B, S, _D_WORD), jnp.float32),
    )(ce, word_vecs, Wb)

    return out
